# baseline (speedup 1.0000x reference)
import functools

import jax
import jax.numpy as jnp
from jax import lax
from jax.experimental import pallas as pl
from jax.experimental.pallas import tpu as pltpu
from jax.experimental.pallas import tpu_sc as plsc

N = 50000
E = 800000
D_IN = 128
H = 64
C = 10
G = 512

F = 32
CH = 400
NCHUNK = 125
RCHUNK = 125
BN = 10000
NB = N // BN
BP = 2000
NP = N // BP


N4 = N // 4
BN4 = 2500
NB4 = N4 // BN4


def _mm1_body(x_ref, w_ref, o_ref):
    xb = x_ref[0]
    outs = [jnp.dot(xb[:, q, :], w_ref[0],
                    preferred_element_type=jnp.float32) for q in range(4)]
    o_ref[0, 0] = jnp.concatenate(outs, axis=1)


def _project1(x, W1s):
    out = pl.pallas_call(
        _mm1_body,
        grid=(NB4, 2),
        in_specs=[
            pl.BlockSpec((1, BN4, 4, D_IN), lambda i, c: (i, 0, 0, 0)),
            pl.BlockSpec((1, D_IN, F), lambda i, c: (c, 0, 0)),
        ],
        out_specs=pl.BlockSpec((1, 1, BN4, 128), lambda i, c: (c, i, 0, 0)),
        out_shape=jax.ShapeDtypeStruct((2, NB4, BN4, 128), jnp.float32),
    )(x.reshape(NB4, BN4, 4, D_IN), W1s)
    return out.reshape(2, N4, 128)


def _mm2_body(a0_ref, a1_ref, b_ref, ab_ref, o_ref):
    cp = pl.program_id(1)
    act0 = jnp.maximum(a0_ref[0, 0] + b_ref[0][None, :], 0.0)
    act1 = jnp.maximum(a1_ref[0, 0] + b_ref[1][None, :], 0.0)
    o_ref[0, 0] = (jnp.dot(act0, ab_ref[0, cp],
                           preferred_element_type=jnp.float32)
                   + jnp.dot(act1, ab_ref[1, cp],
                             preferred_element_type=jnp.float32))


def _project2(agg1p, b1p, Ab):
    a4 = agg1p.reshape(2, NB4, BN4, 128)
    out = pl.pallas_call(
        _mm2_body,
        grid=(NB4, 2),
        in_specs=[
            pl.BlockSpec((1, 1, BN4, 128), lambda i, c: (0, i, 0, 0)),
            pl.BlockSpec((1, 1, BN4, 128), lambda i, c: (1, i, 0, 0)),
            pl.BlockSpec((2, 128), lambda i, c: (0, 0)),
            pl.BlockSpec((2, 2, 128, 128), lambda i, c: (0, 0, 0, 0)),
        ],
        out_specs=pl.BlockSpec((1, 1, BN4, 128), lambda i, c: (c, i, 0, 0)),
        out_shape=jax.ShapeDtypeStruct((2, NB4, BN4, 128), jnp.float32),
    )(a4, a4, b1p, Ab)
    return out.reshape(2, N4, 128)


BPP = 500
NPP = N4 // BPP


def _pool_body(a0_ref, a1_ref, bt_ref, b2_ref, wfc_ref, bfc_ref, o_ref,
               acc_ref):
    i = pl.program_id(0)

    @pl.when(i == 0)
    def _():
        acc_ref[...] = jnp.zeros_like(acc_ref)

    act0 = jnp.maximum(a0_ref[0, 0] + b2_ref[0][None, :], 0.0)
    act1 = jnp.maximum(a1_ref[0, 0] + b2_ref[1][None, :], 0.0)
    ids4 = bt_ref[0]
    gi = lax.broadcasted_iota(jnp.int32, (G, BPP), 0)
    ones = jnp.ones((BPP, 1), jnp.float32)
    total = jnp.zeros_like(acc_ref)
    for q in range(4):
        onehot = (gi == ids4[q][None, :]).astype(jnp.float32)
        hq = jnp.concatenate(
            [act0[:, 32 * q:32 * q + 32], act1[:, 32 * q:32 * q + 32], ones],
            axis=1)
        total += jnp.dot(onehot, hq, preferred_element_type=jnp.float32)
    acc_ref[...] += total

    @pl.when(i == NPP - 1)
    def _():
        sums = acc_ref[:, :H]
        cnts = jnp.maximum(acc_ref[:, H:H + 1], 1.0)
        o_ref[...] = jnp.dot(sums / cnts, wfc_ref[...],
                             preferred_element_type=jnp.float32) + bfc_ref[...]


def _pool_fc(agg2p, batch, b2p, Wfc, bfc):
    batch_t = jnp.transpose(batch.reshape(NPP, BPP, 4), (0, 2, 1))
    a4 = agg2p.reshape(2, NPP, BPP, 128)
    return pl.pallas_call(
        _pool_body,
        grid=(NPP,),
        in_specs=[
            pl.BlockSpec((1, 1, BPP, 128), lambda i: (0, i, 0, 0)),
            pl.BlockSpec((1, 1, BPP, 128), lambda i: (1, i, 0, 0)),
            pl.BlockSpec((1, 4, BPP), lambda i: (i, 0, 0)),
            pl.BlockSpec((2, 128), lambda i: (0, 0)),
            pl.BlockSpec((H, C), lambda i: (0, 0)),
            pl.BlockSpec((1, C), lambda i: (0, 0)),
        ],
        out_specs=pl.BlockSpec((G, C), lambda i: (0, 0)),
        out_shape=jax.ShapeDtypeStruct((G, C), jnp.float32),
        scratch_shapes=[pltpu.VMEM((G, H + 1), jnp.float32)],
    )(a4, a4, batch_t, b2p, Wfc, bfc.reshape(1, C))


def _sc_segsum(p, ed):
    mesh = plsc.VectorSubcoreMesh(core_axis_name="c", subcore_axis_name="s")

    NGRP = NCHUNK // 6

    @functools.partial(
        pl.kernel,
        mesh=mesh,
        out_type=jax.ShapeDtypeStruct((2 * N, F), jnp.float32),
        scratch_types=[
            pltpu.VMEM((3, CH), jnp.int32),
            pltpu.VMEM((3, CH), jnp.int32),
            pltpu.VMEM((3, CH), jnp.int32),
            pltpu.VMEM((CH, F), jnp.float32),
            pltpu.VMEM((CH, F), jnp.float32),
            pltpu.VMEM_SHARED((N, F), jnp.float32),
            pltpu.SemaphoreType.DMA,
            pltpu.SemaphoreType.DMA,
            pltpu.SemaphoreType.DMA,
            pltpu.SemaphoreType.DMA,
            pltpu.SemaphoreType.DMA,
            pltpu.SemaphoreType.DMA,
            pltpu.SemaphoreType.DMA,
        ],
        compiler_params=pltpu.CompilerParams(use_tc_tiling_on_sc=False),
    )
    def k(p_hbm, ed_hbm, out_hbm,
          ed0, ed1, ed2, rows0, rows1,
          acc_sh, semi0, semi1, semi2, semg0, semg1, semsc0, semsc1):
        c = lax.axis_index("c")
        s = lax.axis_index("s")
        ibufs = (ed0, ed1, ed2)
        rbufs = (rows0, rows1)
        semis = (semi0, semi1, semi2)
        semgs = (semg0, semg1)
        semscs = (semsc0, semsc1)

        def start_idx(kk, ib):
            base = 3 * (kk * 16 + s)
            pltpu.async_copy(ed_hbm.at[pl.ds(base, 3)], ibufs[ib], semis[ib])

        def wait_idx(kk, ib):
            base = 3 * (kk * 16 + s)
            pltpu.make_async_copy(
                ed_hbm.at[pl.ds(base, 3)], ibufs[ib], semis[ib]).wait()

        def start_gather(rb, ib):
            pltpu.async_copy(p_hbm.at[c].at[ibufs[ib].at[0]], rbufs[rb],
                             semgs[rb])

        def wait_gather(rb, ib):
            pltpu.make_async_copy(
                p_hbm.at[c].at[ibufs[ib].at[0]], rbufs[rb], semgs[rb]).wait()

        def start_scatter(rb, ib):
            pltpu.async_copy(rbufs[rb], acc_sh.at[ibufs[ib].at[1]],
                             semscs[rb], add=True)

        def wait_scatter(rb, ib):
            pltpu.make_async_copy(
                rbufs[rb], acc_sh.at[ibufs[ib].at[1]], semscs[rb]).wait()

        def scale(rb, ib):
            wbuf = ibufs[ib]
            rbuf = rbufs[rb]

            def body(g, carry):
                w16 = lax.bitcast_convert_type(
                    wbuf[2, pl.ds(g * 16, 16)], jnp.float32)
                for i in range(16):
                    e = g * 16 + i
                    bc = w16.at[jnp.full((16,), i, jnp.int32)].get(
                        mode="promise_in_bounds")
                    rbuf[e, 0:16] = rbuf[e, 0:16] * bc
                    rbuf[e, 16:32] = rbuf[e, 16:32] * bc
                return carry
            lax.fori_loop(0, CH // 16, body, 0)

        def zrow(i, carry):
            rows0[i, 0:16] = jnp.zeros((16,), jnp.float32)
            rows0[i, 16:32] = jnp.zeros((16,), jnp.float32)
            return carry
        lax.fori_loop(0, CH, zrow, 0)

        for jj in range(8):
            j = s + 16 * jj

            @pl.when(j < RCHUNK)
            def _():
                pltpu.sync_copy(rows0, acc_sh.at[pl.ds(j * CH, CH)])

        plsc.subcore_barrier()

        def chunk_step(kk, m2, m3, first=False, pre_g=True, pre_idx=True):
            wait_gather(m2, m3)
            if not first:
                wait_scatter(1 - m2, (m3 + 2) % 3)
            if pre_g:
                wait_idx(kk + 1, (m3 + 1) % 3)
                start_gather(1 - m2, (m3 + 1) % 3)
            scale(m2, m3)
            start_scatter(m2, m3)
            if pre_idx:
                start_idx(kk + 2, (m3 + 2) % 3)

        start_idx(0, 0)
        start_idx(1, 1)
        wait_idx(0, 0)
        start_gather(0, 0)

        chunk_step(0, 0, 0, first=True)
        for j in range(1, 6):
            chunk_step(j, j % 2, j % 3)

        def grp(g, carry):
            k0 = 6 * g
            for j in range(6):
                chunk_step(k0 + j, j % 2, j % 3)
            return carry
        lax.fori_loop(1, NGRP, grp, 0)

        for kk in range(NGRP * 6, NCHUNK):
            chunk_step(kk, kk % 2, kk % 3,
                       pre_g=(kk + 1 < NCHUNK), pre_idx=(kk + 2 < NCHUNK))
        wait_scatter((NCHUNK - 1) % 2, (NCHUNK - 1) % 3)

        plsc.subcore_barrier()

        for jj in range(8):
            j = s + 16 * jj

            @pl.when(j < RCHUNK)
            def _():
                pltpu.sync_copy(acc_sh.at[pl.ds(j * CH, CH)], rows0)
                pltpu.sync_copy(
                    rows0, out_hbm.at[pl.ds(c * N + j * CH, CH)])

    return k(p.reshape(2, N, F), ed)


def kernel(x, edge_index, edge_attr, batch, W1, b1, W2, b2, Wfc, bfc):
    src = edge_index[0]
    dst = edge_index[1]
    W1s = jnp.stack([W1[:, :F], W1[:, F:]])
    eye4 = jnp.eye(4, dtype=jnp.float32)
    Ab = jnp.stack([
        jnp.stack([jnp.kron(eye4, W2[32 * c:32 * c + 32, 32 * cp:32 * cp + 32])
                   for cp in range(2)])
        for c in range(2)])
    b1p = jnp.stack([jnp.tile(b1[:F], 4), jnp.tile(b1[F:], 4)])
    b2p = jnp.stack([jnp.tile(b2[:F], 4), jnp.tile(b2[F:], 4)])
    NT = E // CH
    wbits = lax.bitcast_convert_type(edge_attr, jnp.int32)
    ed = jnp.stack([src.reshape(NT, CH), dst.reshape(NT, CH),
                    wbits.reshape(NT, CH)], axis=1).reshape(3 * NT, CH)
    p1 = _project1(x, W1s)
    agg1 = _sc_segsum(p1.reshape(2 * N, F), ed)
    p2 = _project2(agg1.reshape(2, N4, 128), b1p, Ab)
    agg2 = _sc_segsum(p2.reshape(2 * N, F), ed)
    return _pool_fc(agg2.reshape(2, N4, 128), batch, b2p, Wfc, bfc)


# speedup vs baseline: 1.1293x; 1.1293x over previous
import functools

import jax
import jax.numpy as jnp
from jax import lax
from jax.experimental import pallas as pl
from jax.experimental.pallas import tpu as pltpu
from jax.experimental.pallas import tpu_sc as plsc

N = 50000
E = 800000
D_IN = 128
H = 64
C = 10
G = 512

F = 32
CH = 400
NCHUNK = 125
RCHUNK = 125
BN = 10000
NB = N // BN
BP = 2000
NP = N // BP


N4 = N // 4
BN4 = 2500
NB4 = N4 // BN4


def _mm1_body(x_ref, w_ref, o_ref):
    xb = x_ref[0]
    outs = [jnp.dot(xb[:, q, :], w_ref[0],
                    preferred_element_type=jnp.float32) for q in range(4)]
    o_ref[0, 0] = jnp.concatenate(outs, axis=1)


def _project1(x, W1s):
    out = pl.pallas_call(
        _mm1_body,
        grid=(NB4, 2),
        in_specs=[
            pl.BlockSpec((1, BN4, 4, D_IN), lambda i, c: (i, 0, 0, 0)),
            pl.BlockSpec((1, D_IN, F), lambda i, c: (c, 0, 0)),
        ],
        out_specs=pl.BlockSpec((1, 1, BN4, 128), lambda i, c: (c, i, 0, 0)),
        out_shape=jax.ShapeDtypeStruct((2, NB4, BN4, 128), jnp.float32),
    )(x.reshape(NB4, BN4, 4, D_IN), W1s)
    return out.reshape(2, N4, 128)


def _mm2_body(a0_ref, a1_ref, b_ref, ab_ref, o_ref):
    cp = pl.program_id(1)
    act0 = jnp.maximum(a0_ref[0, 0] + b_ref[0][None, :], 0.0)
    act1 = jnp.maximum(a1_ref[0, 0] + b_ref[1][None, :], 0.0)
    o_ref[0, 0] = (jnp.dot(act0, ab_ref[0, cp],
                           preferred_element_type=jnp.float32)
                   + jnp.dot(act1, ab_ref[1, cp],
                             preferred_element_type=jnp.float32))


def _project2(agg1p, b1p, Ab):
    a4 = agg1p.reshape(2, NB4, BN4, 128)
    out = pl.pallas_call(
        _mm2_body,
        grid=(NB4, 2),
        in_specs=[
            pl.BlockSpec((1, 1, BN4, 128), lambda i, c: (0, i, 0, 0)),
            pl.BlockSpec((1, 1, BN4, 128), lambda i, c: (1, i, 0, 0)),
            pl.BlockSpec((2, 128), lambda i, c: (0, 0)),
            pl.BlockSpec((2, 2, 128, 128), lambda i, c: (0, 0, 0, 0)),
        ],
        out_specs=pl.BlockSpec((1, 1, BN4, 128), lambda i, c: (c, i, 0, 0)),
        out_shape=jax.ShapeDtypeStruct((2, NB4, BN4, 128), jnp.float32),
    )(a4, a4, b1p, Ab)
    return out.reshape(2, N4, 128)


BPP = 500
NPP = N4 // BPP


def _pool_body(a0_ref, a1_ref, bt_ref, b2_ref, wfc_ref, bfc_ref, o_ref,
               acc_ref):
    i = pl.program_id(0)

    @pl.when(i == 0)
    def _():
        acc_ref[...] = jnp.zeros_like(acc_ref)

    act0 = jnp.maximum(a0_ref[0, 0] + b2_ref[0][None, :], 0.0)
    act1 = jnp.maximum(a1_ref[0, 0] + b2_ref[1][None, :], 0.0)
    ids4 = bt_ref[0]
    gi = lax.broadcasted_iota(jnp.int32, (G, BPP), 0)
    ones = jnp.ones((BPP, 1), jnp.float32)
    total = jnp.zeros_like(acc_ref)
    for q in range(4):
        onehot = (gi == ids4[q][None, :]).astype(jnp.float32)
        hq = jnp.concatenate(
            [act0[:, 32 * q:32 * q + 32], act1[:, 32 * q:32 * q + 32], ones],
            axis=1)
        total += jnp.dot(onehot, hq, preferred_element_type=jnp.float32)
    acc_ref[...] += total

    @pl.when(i == NPP - 1)
    def _():
        sums = acc_ref[:, :H]
        cnts = jnp.maximum(acc_ref[:, H:H + 1], 1.0)
        o_ref[...] = jnp.dot(sums / cnts, wfc_ref[...],
                             preferred_element_type=jnp.float32) + bfc_ref[...]


def _pool_fc(agg2p, batch, b2p, Wfc, bfc):
    batch_t = jnp.transpose(batch.reshape(NPP, BPP, 4), (0, 2, 1))
    a4 = agg2p.reshape(2, NPP, BPP, 128)
    return pl.pallas_call(
        _pool_body,
        grid=(NPP,),
        in_specs=[
            pl.BlockSpec((1, 1, BPP, 128), lambda i: (0, i, 0, 0)),
            pl.BlockSpec((1, 1, BPP, 128), lambda i: (1, i, 0, 0)),
            pl.BlockSpec((1, 4, BPP), lambda i: (i, 0, 0)),
            pl.BlockSpec((2, 128), lambda i: (0, 0)),
            pl.BlockSpec((H, C), lambda i: (0, 0)),
            pl.BlockSpec((1, C), lambda i: (0, 0)),
        ],
        out_specs=pl.BlockSpec((G, C), lambda i: (0, 0)),
        out_shape=jax.ShapeDtypeStruct((G, C), jnp.float32),
        scratch_shapes=[pltpu.VMEM((G, H + 1), jnp.float32)],
    )(a4, a4, batch_t, b2p, Wfc, bfc.reshape(1, C))


def _sc_segsum(p, src, dst, w):
    mesh = plsc.VectorSubcoreMesh(core_axis_name="c", subcore_axis_name="s")

    NGRP = NCHUNK // 6

    @functools.partial(
        pl.kernel,
        mesh=mesh,
        out_type=jax.ShapeDtypeStruct((2 * N, F), jnp.float32),
        scratch_types=[
            pltpu.VMEM((CH,), jnp.int32),
            pltpu.VMEM((CH,), jnp.int32),
            pltpu.VMEM((CH,), jnp.float32),
            pltpu.VMEM((CH,), jnp.int32),
            pltpu.VMEM((CH,), jnp.int32),
            pltpu.VMEM((CH,), jnp.float32),
            pltpu.VMEM((CH,), jnp.int32),
            pltpu.VMEM((CH,), jnp.int32),
            pltpu.VMEM((CH,), jnp.float32),
            pltpu.VMEM((CH, F), jnp.float32),
            pltpu.VMEM((CH, F), jnp.float32),
            pltpu.VMEM_SHARED((N, F), jnp.float32),
            pltpu.SemaphoreType.DMA,
            pltpu.SemaphoreType.DMA,
            pltpu.SemaphoreType.DMA,
            pltpu.SemaphoreType.DMA,
            pltpu.SemaphoreType.DMA,
            pltpu.SemaphoreType.DMA,
            pltpu.SemaphoreType.DMA,
        ],
        compiler_params=pltpu.CompilerParams(use_tc_tiling_on_sc=False),
    )
    def k(p_hbm, src_hbm, dst_hbm, w_hbm, out_hbm,
          src0, dst0, w0, src1, dst1, w1, src2, dst2, w2, rows0, rows1,
          acc_sh, semi0, semi1, semi2, semg0, semg1, semsc0, semsc1):
        c = lax.axis_index("c")
        s = lax.axis_index("s")
        ibufs = ((src0, dst0, w0), (src1, dst1, w1), (src2, dst2, w2))
        rbufs = (rows0, rows1)
        semis = (semi0, semi1, semi2)
        semgs = (semg0, semg1)
        semscs = (semsc0, semsc1)

        def start_idx(kk, ib):
            base = (kk * 16 + s) * CH
            sbuf, dbuf, wbuf = ibufs[ib]
            pltpu.async_copy(src_hbm.at[pl.ds(base, CH)], sbuf, semis[ib])
            pltpu.async_copy(dst_hbm.at[pl.ds(base, CH)], dbuf, semis[ib])
            pltpu.async_copy(w_hbm.at[pl.ds(base, CH)], wbuf, semis[ib])

        def wait_idx(kk, ib):
            base = (kk * 16 + s) * CH
            sbuf, dbuf, wbuf = ibufs[ib]
            pltpu.make_async_copy(
                src_hbm.at[pl.ds(base, CH)], sbuf, semis[ib]).wait()
            pltpu.make_async_copy(
                dst_hbm.at[pl.ds(base, CH)], dbuf, semis[ib]).wait()
            pltpu.make_async_copy(
                w_hbm.at[pl.ds(base, CH)], wbuf, semis[ib]).wait()

        def start_gather(rb, ib):
            pltpu.async_copy(p_hbm.at[c].at[ibufs[ib][0]], rbufs[rb],
                             semgs[rb])

        def wait_gather(rb, ib):
            pltpu.make_async_copy(
                p_hbm.at[c].at[ibufs[ib][0]], rbufs[rb], semgs[rb]).wait()

        def start_scatter(rb, ib):
            pltpu.async_copy(rbufs[rb], acc_sh.at[ibufs[ib][1]], semscs[rb],
                             add=True)

        def wait_scatter(rb, ib):
            pltpu.make_async_copy(
                rbufs[rb], acc_sh.at[ibufs[ib][1]], semscs[rb]).wait()

        def scale(rb, ib):
            wbuf = ibufs[ib][2]
            rbuf = rbufs[rb]

            def body(g, carry):
                w16 = wbuf[pl.ds(g * 16, 16)]
                for i in range(16):
                    e = g * 16 + i
                    bc = w16.at[jnp.full((16,), i, jnp.int32)].get(
                        mode="promise_in_bounds")
                    rbuf[e, 0:16] = rbuf[e, 0:16] * bc
                    rbuf[e, 16:32] = rbuf[e, 16:32] * bc
                return carry
            lax.fori_loop(0, CH // 16, body, 0)

        def zrow(i, carry):
            rows0[i, 0:16] = jnp.zeros((16,), jnp.float32)
            rows0[i, 16:32] = jnp.zeros((16,), jnp.float32)
            return carry
        lax.fori_loop(0, CH, zrow, 0)

        for jj in range(8):
            j = s + 16 * jj

            @pl.when(j < RCHUNK)
            def _():
                pltpu.sync_copy(rows0, acc_sh.at[pl.ds(j * CH, CH)])

        plsc.subcore_barrier()

        def chunk_step(kk, m2, m3, first=False, pre_g=True, pre_idx=True):
            wait_gather(m2, m3)
            if not first:
                wait_scatter(1 - m2, (m3 + 2) % 3)
            if pre_g:
                wait_idx(kk + 1, (m3 + 1) % 3)
                start_gather(1 - m2, (m3 + 1) % 3)
            scale(m2, m3)
            start_scatter(m2, m3)
            if pre_idx:
                start_idx(kk + 2, (m3 + 2) % 3)

        start_idx(0, 0)
        start_idx(1, 1)
        wait_idx(0, 0)
        start_gather(0, 0)

        chunk_step(0, 0, 0, first=True)
        for j in range(1, 6):
            chunk_step(j, j % 2, j % 3)

        def grp(g, carry):
            k0 = 6 * g
            for j in range(6):
                chunk_step(k0 + j, j % 2, j % 3)
            return carry
        lax.fori_loop(1, NGRP, grp, 0)

        for kk in range(NGRP * 6, NCHUNK):
            chunk_step(kk, kk % 2, kk % 3,
                       pre_g=(kk + 1 < NCHUNK), pre_idx=(kk + 2 < NCHUNK))
        wait_scatter((NCHUNK - 1) % 2, (NCHUNK - 1) % 3)

        plsc.subcore_barrier()

        for jj in range(8):
            j = s + 16 * jj

            @pl.when(j < RCHUNK)
            def _():
                pltpu.sync_copy(acc_sh.at[pl.ds(j * CH, CH)], rows0)
                pltpu.sync_copy(
                    rows0, out_hbm.at[pl.ds(c * N + j * CH, CH)])

    return k(p.reshape(2, N, F), src, dst, w)


def kernel(x, edge_index, edge_attr, batch, W1, b1, W2, b2, Wfc, bfc):
    src = edge_index[0]
    dst = edge_index[1]
    W1s = jnp.stack([W1[:, :F], W1[:, F:]])
    eye4 = jnp.eye(4, dtype=jnp.float32)
    Ab = jnp.stack([
        jnp.stack([jnp.kron(eye4, W2[32 * c:32 * c + 32, 32 * cp:32 * cp + 32])
                   for cp in range(2)])
        for c in range(2)])
    b1p = jnp.stack([jnp.tile(b1[:F], 4), jnp.tile(b1[F:], 4)])
    b2p = jnp.stack([jnp.tile(b2[:F], 4), jnp.tile(b2[F:], 4)])
    p1 = _project1(x, W1s)
    agg1 = _sc_segsum(p1.reshape(2 * N, F), src, dst, edge_attr)
    p2 = _project2(agg1.reshape(2, N4, 128), b1p, Ab)
    agg2 = _sc_segsum(p2.reshape(2 * N, F), src, dst, edge_attr)
    return _pool_fc(agg2.reshape(2, N4, 128), batch, b2p, Wfc, bfc)


# speedup vs baseline: 1.1824x; 1.0470x over previous
import functools

import jax
import jax.numpy as jnp
from jax import lax
from jax.experimental import pallas as pl
from jax.experimental.pallas import tpu as pltpu
from jax.experimental.pallas import tpu_sc as plsc

N = 50000
E = 800000
D_IN = 128
H = 64
C = 10
G = 512

F = 32
CH = 400
NCHUNK = 125
RCHUNK = 125
BN = 10000
NB = N // BN
BP = 2000
NP = N // BP


N4 = N // 4
BN4 = 2500
NB4 = N4 // BN4


def _mm1_body(x_ref, w_ref, o_ref):
    xb = x_ref[0]
    outs = [jnp.dot(xb[:, q, :], w_ref[0],
                    preferred_element_type=jnp.float32) for q in range(4)]
    o_ref[0, 0] = jnp.concatenate(outs, axis=1)


def _project1(x, W1s):
    out = pl.pallas_call(
        _mm1_body,
        grid=(NB4, 2),
        in_specs=[
            pl.BlockSpec((1, BN4, 4, D_IN), lambda i, c: (i, 0, 0, 0)),
            pl.BlockSpec((1, D_IN, F), lambda i, c: (c, 0, 0)),
        ],
        out_specs=pl.BlockSpec((1, 1, BN4, 128), lambda i, c: (c, i, 0, 0)),
        out_shape=jax.ShapeDtypeStruct((2, NB4, BN4, 128), jnp.float32),
    )(x.reshape(NB4, BN4, 4, D_IN), W1s)
    return out.reshape(2, N4, 128)


def _mm2_body(a0_ref, a1_ref, b_ref, ab_ref, o_ref):
    cp = pl.program_id(1)
    act0 = jnp.maximum(a0_ref[0, 0] + b_ref[0][None, :], 0.0)
    act1 = jnp.maximum(a1_ref[0, 0] + b_ref[1][None, :], 0.0)
    o_ref[0, 0] = (jnp.dot(act0, ab_ref[0, cp],
                           preferred_element_type=jnp.float32)
                   + jnp.dot(act1, ab_ref[1, cp],
                             preferred_element_type=jnp.float32))


def _project2(agg1p, b1p, Ab):
    a4 = agg1p.reshape(2, NB4, BN4, 128)
    out = pl.pallas_call(
        _mm2_body,
        grid=(NB4, 2),
        in_specs=[
            pl.BlockSpec((1, 1, BN4, 128), lambda i, c: (0, i, 0, 0)),
            pl.BlockSpec((1, 1, BN4, 128), lambda i, c: (1, i, 0, 0)),
            pl.BlockSpec((2, 128), lambda i, c: (0, 0)),
            pl.BlockSpec((2, 2, 128, 128), lambda i, c: (0, 0, 0, 0)),
        ],
        out_specs=pl.BlockSpec((1, 1, BN4, 128), lambda i, c: (c, i, 0, 0)),
        out_shape=jax.ShapeDtypeStruct((2, NB4, BN4, 128), jnp.float32),
    )(a4, a4, b1p, Ab)
    return out.reshape(2, N4, 128)


BPP = 500
NPP = N4 // BPP


def _pool_body(a0_ref, a1_ref, bt_ref, b2_ref, wfc_ref, bfc_ref, o_ref,
               acc_ref):
    i = pl.program_id(0)

    @pl.when(i == 0)
    def _():
        acc_ref[...] = jnp.zeros_like(acc_ref)

    act0 = jnp.maximum(a0_ref[0, 0] + b2_ref[0][None, :], 0.0)
    act1 = jnp.maximum(a1_ref[0, 0] + b2_ref[1][None, :], 0.0)
    ids4 = bt_ref[0]
    gi = lax.broadcasted_iota(jnp.int32, (G, BPP), 0)
    ones = jnp.ones((BPP, 1), jnp.float32)
    total = jnp.zeros_like(acc_ref)
    for q in range(4):
        onehot = (gi == ids4[q][None, :]).astype(jnp.float32)
        hq = jnp.concatenate(
            [act0[:, 32 * q:32 * q + 32], act1[:, 32 * q:32 * q + 32], ones],
            axis=1)
        total += jnp.dot(onehot, hq, preferred_element_type=jnp.float32)
    acc_ref[...] += total

    @pl.when(i == NPP - 1)
    def _():
        sums = acc_ref[:, :H]
        cnts = jnp.maximum(acc_ref[:, H:H + 1], 1.0)
        o_ref[...] = jnp.dot(sums / cnts, wfc_ref[...],
                             preferred_element_type=jnp.float32) + bfc_ref[...]


def _pool_fc(agg2p, batch, b2p, Wfc, bfc):
    batch_t = jnp.transpose(batch.reshape(NPP, BPP, 4), (0, 2, 1))
    a4 = agg2p.reshape(2, NPP, BPP, 128)
    return pl.pallas_call(
        _pool_body,
        grid=(NPP,),
        in_specs=[
            pl.BlockSpec((1, 1, BPP, 128), lambda i: (0, i, 0, 0)),
            pl.BlockSpec((1, 1, BPP, 128), lambda i: (1, i, 0, 0)),
            pl.BlockSpec((1, 4, BPP), lambda i: (i, 0, 0)),
            pl.BlockSpec((2, 128), lambda i: (0, 0)),
            pl.BlockSpec((H, C), lambda i: (0, 0)),
            pl.BlockSpec((1, C), lambda i: (0, 0)),
        ],
        out_specs=pl.BlockSpec((G, C), lambda i: (0, 0)),
        out_shape=jax.ShapeDtypeStruct((G, C), jnp.float32),
        scratch_shapes=[pltpu.VMEM((G, H + 1), jnp.float32)],
    )(a4, a4, batch_t, b2p, Wfc, bfc.reshape(1, C))


def _sc_segsum(p, ei, w):
    mesh = plsc.VectorSubcoreMesh(core_axis_name="c", subcore_axis_name="s")

    NGRP = NCHUNK // 6

    @functools.partial(
        pl.kernel,
        mesh=mesh,
        out_type=jax.ShapeDtypeStruct((2 * N, F), jnp.float32),
        scratch_types=[
            pltpu.VMEM((CH,), jnp.int32),
            pltpu.VMEM((CH,), jnp.int32),
            pltpu.VMEM((CH,), jnp.float32),
            pltpu.VMEM((CH,), jnp.int32),
            pltpu.VMEM((CH,), jnp.int32),
            pltpu.VMEM((CH,), jnp.float32),
            pltpu.VMEM((CH,), jnp.int32),
            pltpu.VMEM((CH,), jnp.int32),
            pltpu.VMEM((CH,), jnp.float32),
            pltpu.VMEM((CH, F), jnp.float32),
            pltpu.VMEM((CH, F), jnp.float32),
            pltpu.VMEM_SHARED((N, F), jnp.float32),
            pltpu.SemaphoreType.DMA,
            pltpu.SemaphoreType.DMA,
            pltpu.SemaphoreType.DMA,
            pltpu.SemaphoreType.DMA,
            pltpu.SemaphoreType.DMA,
            pltpu.SemaphoreType.DMA,
            pltpu.SemaphoreType.DMA,
        ],
        compiler_params=pltpu.CompilerParams(use_tc_tiling_on_sc=False),
    )
    def k(p_hbm, ei_hbm, w_hbm, out_hbm,
          src0, dst0, w0, src1, dst1, w1, src2, dst2, w2, rows0, rows1,
          acc_sh, semi0, semi1, semi2, semg0, semg1, semsc0, semsc1):
        c = lax.axis_index("c")
        s = lax.axis_index("s")
        src_hbm = ei_hbm.at[0]
        dst_hbm = ei_hbm.at[1]
        ibufs = ((src0, dst0, w0), (src1, dst1, w1), (src2, dst2, w2))
        rbufs = (rows0, rows1)
        semis = (semi0, semi1, semi2)
        semgs = (semg0, semg1)
        semscs = (semsc0, semsc1)

        def start_idx(kk, ib):
            base = (kk * 16 + s) * CH
            sbuf, dbuf, wbuf = ibufs[ib]
            pltpu.async_copy(src_hbm.at[pl.ds(base, CH)], sbuf, semis[ib])
            pltpu.async_copy(dst_hbm.at[pl.ds(base, CH)], dbuf, semis[ib])
            pltpu.async_copy(w_hbm.at[pl.ds(base, CH)], wbuf, semis[ib])

        def wait_idx(kk, ib):
            base = (kk * 16 + s) * CH
            sbuf, dbuf, wbuf = ibufs[ib]
            pltpu.make_async_copy(
                src_hbm.at[pl.ds(base, CH)], sbuf, semis[ib]).wait()
            pltpu.make_async_copy(
                dst_hbm.at[pl.ds(base, CH)], dbuf, semis[ib]).wait()
            pltpu.make_async_copy(
                w_hbm.at[pl.ds(base, CH)], wbuf, semis[ib]).wait()

        def start_gather(rb, ib):
            pltpu.async_copy(p_hbm.at[c].at[ibufs[ib][0]], rbufs[rb],
                             semgs[rb])

        def wait_gather(rb, ib):
            pltpu.make_async_copy(
                p_hbm.at[c].at[ibufs[ib][0]], rbufs[rb], semgs[rb]).wait()

        def start_scatter(rb, ib):
            pltpu.async_copy(rbufs[rb], acc_sh.at[ibufs[ib][1]], semscs[rb],
                             add=True)

        def wait_scatter(rb, ib):
            pltpu.make_async_copy(
                rbufs[rb], acc_sh.at[ibufs[ib][1]], semscs[rb]).wait()

        def scale(rb, ib):
            wbuf = ibufs[ib][2]
            rbuf = rbufs[rb]

            def body(g, carry):
                w16 = wbuf[pl.ds(g * 16, 16)]
                for i in range(16):
                    e = g * 16 + i
                    bc = w16.at[jnp.full((16,), i, jnp.int32)].get(
                        mode="promise_in_bounds")
                    rbuf[e, 0:16] = rbuf[e, 0:16] * bc
                    rbuf[e, 16:32] = rbuf[e, 16:32] * bc
                return carry
            lax.fori_loop(0, CH // 16, body, 0)

        def zrow(i, carry):
            rows0[i, 0:16] = jnp.zeros((16,), jnp.float32)
            rows0[i, 16:32] = jnp.zeros((16,), jnp.float32)
            return carry
        lax.fori_loop(0, CH, zrow, 0)

        for jj in range(8):
            j = s + 16 * jj

            @pl.when(j < RCHUNK)
            def _():
                pltpu.sync_copy(rows0, acc_sh.at[pl.ds(j * CH, CH)])

        plsc.subcore_barrier()

        def chunk_step(kk, m2, m3, first=False, pre_g=True, pre_idx=True):
            wait_gather(m2, m3)
            if not first:
                wait_scatter(1 - m2, (m3 + 2) % 3)
            if pre_g:
                wait_idx(kk + 1, (m3 + 1) % 3)
                start_gather(1 - m2, (m3 + 1) % 3)
            scale(m2, m3)
            start_scatter(m2, m3)
            if pre_idx:
                start_idx(kk + 2, (m3 + 2) % 3)

        start_idx(0, 0)
        start_idx(1, 1)
        wait_idx(0, 0)
        start_gather(0, 0)

        chunk_step(0, 0, 0, first=True)
        for j in range(1, 6):
            chunk_step(j, j % 2, j % 3)

        def grp(g, carry):
            k0 = 6 * g
            for j in range(6):
                chunk_step(k0 + j, j % 2, j % 3)
            return carry
        lax.fori_loop(1, NGRP, grp, 0)

        for kk in range(NGRP * 6, NCHUNK):
            chunk_step(kk, kk % 2, kk % 3,
                       pre_g=(kk + 1 < NCHUNK), pre_idx=(kk + 2 < NCHUNK))
        wait_scatter((NCHUNK - 1) % 2, (NCHUNK - 1) % 3)

        plsc.subcore_barrier()

        for jj in range(8):
            j = s + 16 * jj

            @pl.when(j < RCHUNK)
            def _():
                pltpu.sync_copy(acc_sh.at[pl.ds(j * CH, CH)], rows0)
                pltpu.sync_copy(
                    rows0, out_hbm.at[pl.ds(c * N + j * CH, CH)])

    return k(p.reshape(2, N, F), ei, w)


def kernel(x, edge_index, edge_attr, batch, W1, b1, W2, b2, Wfc, bfc):
    src = edge_index[0]
    dst = edge_index[1]
    W1s = jnp.stack([W1[:, :F], W1[:, F:]])
    eye4 = jnp.eye(4, dtype=jnp.float32)
    Ab = jnp.stack([
        jnp.stack([jnp.kron(eye4, W2[32 * c:32 * c + 32, 32 * cp:32 * cp + 32])
                   for cp in range(2)])
        for c in range(2)])
    b1p = jnp.stack([jnp.tile(b1[:F], 4), jnp.tile(b1[F:], 4)])
    b2p = jnp.stack([jnp.tile(b2[:F], 4), jnp.tile(b2[F:], 4)])
    p1 = _project1(x, W1s)
    agg1 = _sc_segsum(p1.reshape(2 * N, F), edge_index, edge_attr)
    p2 = _project2(agg1.reshape(2, N4, 128), b1p, Ab)
    agg2 = _sc_segsum(p2.reshape(2 * N, F), edge_index, edge_attr)
    return _pool_fc(agg2.reshape(2, N4, 128), batch, b2p, Wfc, bfc)


# speedup vs baseline: 1.1993x; 1.0144x over previous
import functools

import jax
import jax.numpy as jnp
from jax import lax
from jax.experimental import pallas as pl
from jax.experimental.pallas import tpu as pltpu
from jax.experimental.pallas import tpu_sc as plsc

N = 50000
E = 800000
D_IN = 128
H = 64
C = 10
G = 512

F = 32
CH = 400
NCHUNK = 125
RCHUNK = 125
BN = 10000
NB = N // BN
BP = 2000
NP = N // BP


N4 = N // 4
BN4 = 2500
NB4 = N4 // BN4


def _mm1_body(x_ref, w_ref, o_ref):
    xb = x_ref[0]
    outs = [jnp.dot(xb[:, q, :], w_ref[0],
                    preferred_element_type=jnp.float32) for q in range(4)]
    o_ref[0, 0] = jnp.concatenate(outs, axis=1)


def _project1(x, W1s):
    out = pl.pallas_call(
        _mm1_body,
        grid=(NB4, 2),
        in_specs=[
            pl.BlockSpec((1, BN4, 4, D_IN), lambda i, c: (i, 0, 0, 0)),
            pl.BlockSpec((1, D_IN, F), lambda i, c: (c, 0, 0)),
        ],
        out_specs=pl.BlockSpec((1, 1, BN4, 128), lambda i, c: (c, i, 0, 0)),
        out_shape=jax.ShapeDtypeStruct((2, NB4, BN4, 128), jnp.float32),
    )(x.reshape(NB4, BN4, 4, D_IN), W1s)
    return out.reshape(2, N4, 128)


def _mm2_body(a0_ref, a1_ref, b_ref, ab_ref, o_ref):
    cp = pl.program_id(1)
    act0 = jnp.maximum(a0_ref[0, 0] + b_ref[0][None, :], 0.0)
    act1 = jnp.maximum(a1_ref[0, 0] + b_ref[1][None, :], 0.0)
    o_ref[0, 0] = (jnp.dot(act0, ab_ref[0, cp],
                           preferred_element_type=jnp.float32)
                   + jnp.dot(act1, ab_ref[1, cp],
                             preferred_element_type=jnp.float32))


def _project2(agg1p, b1p, Ab):
    a4 = agg1p.reshape(2, NB4, BN4, 128)
    out = pl.pallas_call(
        _mm2_body,
        grid=(NB4, 2),
        in_specs=[
            pl.BlockSpec((1, 1, BN4, 128), lambda i, c: (0, i, 0, 0)),
            pl.BlockSpec((1, 1, BN4, 128), lambda i, c: (1, i, 0, 0)),
            pl.BlockSpec((2, 128), lambda i, c: (0, 0)),
            pl.BlockSpec((2, 2, 128, 128), lambda i, c: (0, 0, 0, 0)),
        ],
        out_specs=pl.BlockSpec((1, 1, BN4, 128), lambda i, c: (c, i, 0, 0)),
        out_shape=jax.ShapeDtypeStruct((2, NB4, BN4, 128), jnp.float32),
    )(a4, a4, b1p, Ab)
    return out.reshape(2, N4, 128)


BPP = 500
NPP = N4 // BPP


def _pool_body(a0_ref, a1_ref, bt_ref, b2_ref, wfc_ref, bfc_ref, o_ref,
               acc_ref):
    i = pl.program_id(0)

    @pl.when(i == 0)
    def _():
        acc_ref[...] = jnp.zeros_like(acc_ref)

    act0 = jnp.maximum(a0_ref[0, 0] + b2_ref[0][None, :], 0.0)
    act1 = jnp.maximum(a1_ref[0, 0] + b2_ref[1][None, :], 0.0)
    ids4 = bt_ref[0]
    gi = lax.broadcasted_iota(jnp.int32, (G, BPP), 0)
    ones = jnp.ones((BPP, 1), jnp.float32)
    total = jnp.zeros_like(acc_ref)
    for q in range(4):
        onehot = (gi == ids4[q][None, :]).astype(jnp.float32)
        hq = jnp.concatenate(
            [act0[:, 32 * q:32 * q + 32], act1[:, 32 * q:32 * q + 32], ones],
            axis=1)
        total += jnp.dot(onehot, hq, preferred_element_type=jnp.float32)
    acc_ref[...] += total

    @pl.when(i == NPP - 1)
    def _():
        sums = acc_ref[:, :H]
        cnts = jnp.maximum(acc_ref[:, H:H + 1], 1.0)
        o_ref[...] = jnp.dot(sums / cnts, wfc_ref[...],
                             preferred_element_type=jnp.float32) + bfc_ref[...]


def _pool_fc(agg2p, batch, b2p, Wfc, bfc):
    batch_t = jnp.transpose(batch.reshape(NPP, BPP, 4), (0, 2, 1))
    a4 = agg2p.reshape(2, NPP, BPP, 128)
    return pl.pallas_call(
        _pool_body,
        grid=(NPP,),
        in_specs=[
            pl.BlockSpec((1, 1, BPP, 128), lambda i: (0, i, 0, 0)),
            pl.BlockSpec((1, 1, BPP, 128), lambda i: (1, i, 0, 0)),
            pl.BlockSpec((1, 4, BPP), lambda i: (i, 0, 0)),
            pl.BlockSpec((2, 128), lambda i: (0, 0)),
            pl.BlockSpec((H, C), lambda i: (0, 0)),
            pl.BlockSpec((1, C), lambda i: (0, 0)),
        ],
        out_specs=pl.BlockSpec((G, C), lambda i: (0, 0)),
        out_shape=jax.ShapeDtypeStruct((G, C), jnp.float32),
        scratch_shapes=[pltpu.VMEM((G, H + 1), jnp.float32)],
    )(a4, a4, batch_t, b2p, Wfc, bfc.reshape(1, C))


def _sc_segsum(p, ei, w):
    mesh = plsc.VectorSubcoreMesh(core_axis_name="c", subcore_axis_name="s")

    NGRP = NCHUNK // 6

    @functools.partial(
        pl.kernel,
        mesh=mesh,
        out_type=jax.ShapeDtypeStruct((2 * N, F), jnp.float32),
        scratch_types=[
            pltpu.VMEM((CH,), jnp.int32),
            pltpu.VMEM((CH,), jnp.int32),
            pltpu.VMEM((CH,), jnp.float32),
            pltpu.VMEM((CH,), jnp.int32),
            pltpu.VMEM((CH,), jnp.int32),
            pltpu.VMEM((CH,), jnp.float32),
            pltpu.VMEM((CH,), jnp.int32),
            pltpu.VMEM((CH,), jnp.int32),
            pltpu.VMEM((CH,), jnp.float32),
            pltpu.VMEM((CH, F), jnp.float32),
            pltpu.VMEM((CH, F), jnp.float32),
            pltpu.VMEM_SHARED((N, F), jnp.float32),
            pltpu.SemaphoreType.DMA,
            pltpu.SemaphoreType.DMA,
            pltpu.SemaphoreType.DMA,
            pltpu.SemaphoreType.DMA,
            pltpu.SemaphoreType.DMA,
            pltpu.SemaphoreType.DMA,
            pltpu.SemaphoreType.DMA,
            pltpu.SemaphoreType.DMA,
        ],
        compiler_params=pltpu.CompilerParams(use_tc_tiling_on_sc=False),
    )
    def k(p_hbm, ei_hbm, w_hbm, out_hbm,
          src0, dst0, w0, src1, dst1, w1, src2, dst2, w2, rows0, rows1,
          acc_sh, semi0, semi1, semi2, semg0, semg1, semsc0, semsc1,
          semz):
        c = lax.axis_index("c")
        s = lax.axis_index("s")
        src_hbm = ei_hbm.at[0]
        dst_hbm = ei_hbm.at[1]
        ibufs = ((src0, dst0, w0), (src1, dst1, w1), (src2, dst2, w2))
        rbufs = (rows0, rows1)
        semis = (semi0, semi1, semi2)
        semgs = (semg0, semg1)
        semscs = (semsc0, semsc1)

        def start_idx(kk, ib):
            base = (kk * 16 + s) * CH
            sbuf, dbuf, wbuf = ibufs[ib]
            pltpu.async_copy(src_hbm.at[pl.ds(base, CH)], sbuf, semis[ib])
            pltpu.async_copy(dst_hbm.at[pl.ds(base, CH)], dbuf, semis[ib])
            pltpu.async_copy(w_hbm.at[pl.ds(base, CH)], wbuf, semis[ib])

        def wait_idx(kk, ib):
            base = (kk * 16 + s) * CH
            sbuf, dbuf, wbuf = ibufs[ib]
            pltpu.make_async_copy(
                src_hbm.at[pl.ds(base, CH)], sbuf, semis[ib]).wait()
            pltpu.make_async_copy(
                dst_hbm.at[pl.ds(base, CH)], dbuf, semis[ib]).wait()
            pltpu.make_async_copy(
                w_hbm.at[pl.ds(base, CH)], wbuf, semis[ib]).wait()

        def start_gather(rb, ib):
            pltpu.async_copy(p_hbm.at[c].at[ibufs[ib][0]], rbufs[rb],
                             semgs[rb])

        def wait_gather(rb, ib):
            pltpu.make_async_copy(
                p_hbm.at[c].at[ibufs[ib][0]], rbufs[rb], semgs[rb]).wait()

        def start_scatter(rb, ib):
            pltpu.async_copy(rbufs[rb], acc_sh.at[ibufs[ib][1]], semscs[rb],
                             add=True)

        def wait_scatter(rb, ib):
            pltpu.make_async_copy(
                rbufs[rb], acc_sh.at[ibufs[ib][1]], semscs[rb]).wait()

        def scale(rb, ib):
            wbuf = ibufs[ib][2]
            rbuf = rbufs[rb]

            def body(g, carry):
                w16 = wbuf[pl.ds(g * 16, 16)]
                for i in range(16):
                    e = g * 16 + i
                    bc = w16.at[jnp.full((16,), i, jnp.int32)].get(
                        mode="promise_in_bounds")
                    rbuf[e, 0:16] = rbuf[e, 0:16] * bc
                    rbuf[e, 16:32] = rbuf[e, 16:32] * bc
                return carry
            lax.fori_loop(0, CH // 16, body, 0)

        start_idx(0, 0)
        start_idx(1, 1)

        def zrow(i, carry):
            rows0[i, 0:16] = jnp.zeros((16,), jnp.float32)
            rows0[i, 16:32] = jnp.zeros((16,), jnp.float32)
            return carry
        lax.fori_loop(0, CH, zrow, 0)

        for jj in range(8):
            j = s + 16 * jj

            @pl.when(j < RCHUNK)
            def _():
                pltpu.async_copy(rows0, acc_sh.at[pl.ds(j * CH, CH)], semz)
        for jj in range(8):
            j = s + 16 * jj

            @pl.when(j < RCHUNK)
            def _():
                pltpu.make_async_copy(
                    rows0, acc_sh.at[pl.ds(j * CH, CH)], semz).wait()

        plsc.subcore_barrier()

        def chunk_step(kk, m2, m3, first=False, pre_g=True, pre_idx=True):
            wait_gather(m2, m3)
            if not first:
                wait_scatter(1 - m2, (m3 + 2) % 3)
            if pre_g:
                wait_idx(kk + 1, (m3 + 1) % 3)
                start_gather(1 - m2, (m3 + 1) % 3)
            scale(m2, m3)
            start_scatter(m2, m3)
            if pre_idx:
                start_idx(kk + 2, (m3 + 2) % 3)

        wait_idx(0, 0)
        start_gather(0, 0)

        chunk_step(0, 0, 0, first=True)
        for j in range(1, 6):
            chunk_step(j, j % 2, j % 3)

        def grp(g, carry):
            k0 = 6 * g
            for j in range(6):
                chunk_step(k0 + j, j % 2, j % 3)
            return carry
        lax.fori_loop(1, NGRP, grp, 0)

        for kk in range(NGRP * 6, NCHUNK):
            chunk_step(kk, kk % 2, kk % 3,
                       pre_g=(kk + 1 < NCHUNK), pre_idx=(kk + 2 < NCHUNK))
        wait_scatter((NCHUNK - 1) % 2, (NCHUNK - 1) % 3)

        plsc.subcore_barrier()

        for jj in range(8):
            j = s + 16 * jj
            rb = rbufs[jj % 2]
            sg = semgs[jj % 2]

            @pl.when(j < RCHUNK)
            def _():
                pltpu.async_copy(acc_sh.at[pl.ds(j * CH, CH)], rb, sg)
            if jj >= 1:
                jp = s + 16 * (jj - 1)
                rbp = rbufs[(jj - 1) % 2]
                sgp = semgs[(jj - 1) % 2]

                @pl.when(jp < RCHUNK)
                def _():
                    pltpu.make_async_copy(
                        acc_sh.at[pl.ds(jp * CH, CH)], rbp, sgp).wait()
                    pltpu.sync_copy(
                        rbp, out_hbm.at[pl.ds(c * N + jp * CH, CH)])
        jp = s + 16 * 7
        rbp = rbufs[7 % 2]
        sgp = semgs[7 % 2]

        @pl.when(jp < RCHUNK)
        def _():
            pltpu.make_async_copy(
                acc_sh.at[pl.ds(jp * CH, CH)], rbp, sgp).wait()
            pltpu.sync_copy(
                rbp, out_hbm.at[pl.ds(c * N + jp * CH, CH)])

    return k(p.reshape(2, N, F), ei, w)


def kernel(x, edge_index, edge_attr, batch, W1, b1, W2, b2, Wfc, bfc):
    src = edge_index[0]
    dst = edge_index[1]
    W1s = jnp.stack([W1[:, :F], W1[:, F:]])
    eye4 = jnp.eye(4, dtype=jnp.float32)
    Ab = jnp.stack([
        jnp.stack([jnp.kron(eye4, W2[32 * c:32 * c + 32, 32 * cp:32 * cp + 32])
                   for cp in range(2)])
        for c in range(2)])
    b1p = jnp.stack([jnp.tile(b1[:F], 4), jnp.tile(b1[F:], 4)])
    b2p = jnp.stack([jnp.tile(b2[:F], 4), jnp.tile(b2[F:], 4)])
    p1 = _project1(x, W1s)
    agg1 = _sc_segsum(p1.reshape(2 * N, F), edge_index, edge_attr)
    p2 = _project2(agg1.reshape(2, N4, 128), b1p, Ab)
    agg2 = _sc_segsum(p2.reshape(2 * N, F), edge_index, edge_attr)
    return _pool_fc(agg2.reshape(2, N4, 128), batch, b2p, Wfc, bfc)
